# baseline (speedup 1.0000x reference)
import functools

import jax
import jax.numpy as jnp
from jax.experimental import pallas as pl
from jax.experimental.pallas import tpu as pltpu

_B, _L, _D, _O, _E, _T = 4, 2048, 1024, 1024, 8, 256
_N = _L // _T
_S = _B * _N
_SB = 4
_TP = 128
_KP = _S // _SB
_OO = 256
_OH = _O // 2
_NM = _N - 1


def _prep_body(x_ref, wg_ref, w_ref, rw_ref, b_ref, rb_ref,
               gates_ref, loss_ref, xbf_ref, wt_ref, dbm_ref, logits_ref):
    k = pl.program_id(0)
    w = w_ref[...]
    rw = rw_ref[...]
    dw = w - rw[None]
    sdw = jnp.sum(dw, axis=0)
    dwm = jnp.where(dw * sdw[None] > 0, jnp.abs(dw), 0.0)
    wt_ref[...] = (rw[None] + dwm).astype(jnp.bfloat16)
    db = b_ref[...] - rb_ref[...]
    sdb = jnp.sum(db, axis=0, keepdims=True)
    dbm_ref[...] = jnp.where(db * sdb > 0, jnp.abs(db), 0.0)

    xbf_ref[...] = x_ref[...].astype(jnp.bfloat16)
    xm = jnp.mean(x_ref[...], axis=1)
    logits_ref[k] = jax.lax.dot_general(
        xm, wg_ref[...], (((1,), (0,)), ((), ())),
        preferred_element_type=jnp.float32)

    @pl.when(k == pl.num_programs(0) - 1)
    def _():
        logits = logits_ref[...].reshape(_S, _E)
        m = jnp.max(logits, axis=1, keepdims=True)
        ex = jnp.exp(logits - m)
        p = ex / jnp.sum(ex, axis=1, keepdims=True)
        imp = jnp.sum(p, axis=0, keepdims=True)
        ld = jnp.sum((p > 0).astype(jnp.float32), axis=0, keepdims=True)

        def cv2(v):
            mean = jnp.sum(v, axis=1, keepdims=True) / _E
            var = jnp.sum((v - mean) ** 2, axis=1, keepdims=True) / (_E - 1)
            return var / (mean * mean + 1e-10)

        loss_ref[...] = (cv2(imp) + cv2(ld)) * 0.001
        rolled = jnp.concatenate([p[:1], p[:-1]], axis=0)
        row = jax.lax.broadcasted_iota(jnp.int32, (_S, _E), 0)
        gates_ref[...] = jnp.where(row % _N == 0, p, rolled)


def _moe_body(g_ref, x_ref, wt_ref, dbm_ref, rb_ref, out_ref):
    b = pl.program_id(1)

    g = [[g_ref[_N * b + (0 if i == 0 else i + 1), e] for e in range(_E)]
         for i in range(_NM)]
    gbf = [[v.astype(jnp.bfloat16) for v in row] for row in g]

    def xs(i):
        if i == 0:
            return x_ref[pl.ds(0, 2)].reshape(2 * _T, _D)
        return x_ref[i + 1]

    def treesum(terms):
        while len(terms) > 1:
            terms = [terms[j] + terms[j + 1] for j in range(0, len(terms), 2)]
        return terms[0]

    mrows = []
    for i in range(_NM):
        mrows.append(rb_ref[...] + treesum(
            [g[i][e] * dbm_ref[pl.ds(e, 1), :] for e in range(_E)]))
    pieces = []
    for oo in range(_OH // _OO):
        sl = pl.ds(oo * _OO, _OO)
        we = [wt_ref[e, sl, :] for e in range(_E)]
        ys = []
        for i in range(_NM):
            merged = treesum([gbf[i][e] * we[e] for e in range(_E)])
            ys.append(jax.lax.dot_general(
                xs(i), merged, (((1,), (1,)), ((), ())),
                preferred_element_type=jnp.float32))
        pieces.append(jnp.concatenate(ys, axis=0))
    y = jnp.concatenate(pieces, axis=1)
    bias_full = jnp.concatenate([mrows[0]] * 2 + mrows[1:], axis=0)
    out_ref[...] = y.reshape(_N, _T, _OH) + bias_full[:, None, :]


def _build_calls(interpret=False):
    prep = pl.pallas_call(
        _prep_body,
        grid=(_KP,),
        in_specs=[
            pl.BlockSpec((_SB, _T, _D), lambda k: (k, 0, 0)),
            pl.BlockSpec((_D, _E), lambda k: (0, 0)),
            pl.BlockSpec((_E, _TP, _D), lambda k: (0, k, 0)),
            pl.BlockSpec((_TP, _D), lambda k: (k, 0)),
            pl.BlockSpec((_E, _TP), lambda k: (0, k)),
            pl.BlockSpec((1, _TP), lambda k: (0, k)),
        ],
        out_specs=[
            pl.BlockSpec((_S, _E), lambda k: (0, 0)),
            pl.BlockSpec((1, 1), lambda k: (0, 0)),
            pl.BlockSpec((_SB, _T, _D), lambda k: (k, 0, 0)),
            pl.BlockSpec((_E, _TP, _D), lambda k: (0, k, 0)),
            pl.BlockSpec((_E, _TP), lambda k: (0, k)),
        ],
        out_shape=[
            jax.ShapeDtypeStruct((_S, _E), jnp.float32),
            jax.ShapeDtypeStruct((1, 1), jnp.float32),
            jax.ShapeDtypeStruct((_S, _T, _D), jnp.bfloat16),
            jax.ShapeDtypeStruct((_E, _O, _D), jnp.bfloat16),
            jax.ShapeDtypeStruct((_E, _O), jnp.float32),
        ],
        scratch_shapes=[pltpu.VMEM((_KP, _SB, _E), jnp.float32)],
        interpret=interpret,
    )
    moe = pl.pallas_call(
        _moe_body,
        grid=(_O // _OH, _B),
        in_specs=[
            pl.BlockSpec(memory_space=pltpu.SMEM),
            pl.BlockSpec((_N, _T, _D), lambda oh, b: (b, 0, 0)),
            pl.BlockSpec((_E, _OH, _D), lambda oh, b: (0, oh, 0)),
            pl.BlockSpec((_E, _OH), lambda oh, b: (0, oh)),
            pl.BlockSpec((1, _OH), lambda oh, b: (0, oh)),
        ],
        out_specs=pl.BlockSpec((_N, _T, _OH), lambda oh, b: (b, 0, oh)),
        out_shape=jax.ShapeDtypeStruct((_S, _T, _O), jnp.float32),
        interpret=interpret,
    )
    return prep, moe


_PREP, _MOE = _build_calls()


def kernel(x, w_gate, weight, bias, res_weight, res_bias):
    xc = x.reshape(_S, _T, _D)
    gates, loss, xbf, wt, dbm = _PREP(xc, w_gate, weight, res_weight,
                                      bias, res_bias)
    out = _MOE(gates, xbf, wt, dbm, res_bias)
    return out.reshape(_B, _L, _O), loss[0, 0]


# speedup vs baseline: 1.0342x; 1.0342x over previous
import functools

import jax
import jax.numpy as jnp
from jax.experimental import pallas as pl
from jax.experimental.pallas import tpu as pltpu

_B, _L, _D, _O, _E, _T = 4, 2048, 1024, 1024, 8, 256
_N = _L // _T
_S = _B * _N
_SB = 4
_TP = 128
_KP = _S // _SB
_OO = 256
_OH = _O // 2
_NM = _N - 1


def _prep_body(x_ref, wg_ref, w_ref, rw_ref, b_ref, rb_ref,
               gates_ref, loss_ref, wt_ref, dbm_ref, logits_ref):
    k = pl.program_id(0)
    w = w_ref[...]
    rw = rw_ref[...]
    dw = w - rw[None]
    sdw = jnp.sum(dw, axis=0)
    dwm = jnp.where(dw * sdw[None] > 0, jnp.abs(dw), 0.0)
    wt_ref[...] = (rw[None] + dwm).astype(jnp.bfloat16)
    db = b_ref[...] - rb_ref[...]
    sdb = jnp.sum(db, axis=0, keepdims=True)
    dbm_ref[...] = jnp.where(db * sdb > 0, jnp.abs(db), 0.0)

    xm = jnp.mean(x_ref[...], axis=1)
    logits_ref[k] = jax.lax.dot_general(
        xm, wg_ref[...], (((1,), (0,)), ((), ())),
        preferred_element_type=jnp.float32)

    @pl.when(k == pl.num_programs(0) - 1)
    def _():
        logits = logits_ref[...].reshape(_S, _E)
        m = jnp.max(logits, axis=1, keepdims=True)
        ex = jnp.exp(logits - m)
        p = ex / jnp.sum(ex, axis=1, keepdims=True)
        imp = jnp.sum(p, axis=0, keepdims=True)
        ld = jnp.sum((p > 0).astype(jnp.float32), axis=0, keepdims=True)

        def cv2(v):
            mean = jnp.sum(v, axis=1, keepdims=True) / _E
            var = jnp.sum((v - mean) ** 2, axis=1, keepdims=True) / (_E - 1)
            return var / (mean * mean + 1e-10)

        loss_ref[...] = (cv2(imp) + cv2(ld)) * 0.001
        rolled = jnp.concatenate([p[:1], p[:-1]], axis=0)
        row = jax.lax.broadcasted_iota(jnp.int32, (_S, _E), 0)
        gates_ref[...] = jnp.where(row % _N == 0, p, rolled)


def _moe_body(g_ref, x_ref, wt_ref, dbm_ref, rb_ref, out_ref):
    b = pl.program_id(1)

    g = [[g_ref[_N * b + (0 if i == 0 else i + 1), e] for e in range(_E)]
         for i in range(_NM)]
    gbf = [[v.astype(jnp.bfloat16) for v in row] for row in g]

    def xs(i):
        if i == 0:
            return x_ref[pl.ds(0, 2)].reshape(2 * _T, _D).astype(jnp.bfloat16)
        return x_ref[i + 1].astype(jnp.bfloat16)

    def treesum(terms):
        while len(terms) > 1:
            terms = [terms[j] + terms[j + 1] for j in range(0, len(terms), 2)]
        return terms[0]

    mrows = []
    for i in range(_NM):
        mrows.append(rb_ref[...] + treesum(
            [g[i][e] * dbm_ref[pl.ds(e, 1), :] for e in range(_E)]))
    pieces = []
    for oo in range(_OH // _OO):
        sl = pl.ds(oo * _OO, _OO)
        we = [wt_ref[e, sl, :] for e in range(_E)]
        ys = []
        for i in range(_NM):
            merged = gbf[i][0] * we[0]
            for e in range(1, _E):
                merged = merged + gbf[i][e] * we[e]
            ys.append(jax.lax.dot_general(
                xs(i), merged, (((1,), (1,)), ((), ())),
                preferred_element_type=jnp.float32))
        pieces.append(jnp.concatenate(ys, axis=0))
    y = jnp.concatenate(pieces, axis=1)
    bias_full = jnp.concatenate([mrows[0]] * 2 + mrows[1:], axis=0)
    out_ref[...] = y.reshape(_N, _T, _OH) + bias_full[:, None, :]


def _build_calls(interpret=False):
    prep = pl.pallas_call(
        _prep_body,
        grid=(_KP,),
        in_specs=[
            pl.BlockSpec((_SB, _T, _D), lambda k: (k, 0, 0)),
            pl.BlockSpec((_D, _E), lambda k: (0, 0)),
            pl.BlockSpec((_E, _TP, _D), lambda k: (0, k, 0)),
            pl.BlockSpec((_TP, _D), lambda k: (k, 0)),
            pl.BlockSpec((_E, _TP), lambda k: (0, k)),
            pl.BlockSpec((1, _TP), lambda k: (0, k)),
        ],
        out_specs=[
            pl.BlockSpec((_S, _E), lambda k: (0, 0)),
            pl.BlockSpec((1, 1), lambda k: (0, 0)),
            pl.BlockSpec((_E, _TP, _D), lambda k: (0, k, 0)),
            pl.BlockSpec((_E, _TP), lambda k: (0, k)),
        ],
        out_shape=[
            jax.ShapeDtypeStruct((_S, _E), jnp.float32),
            jax.ShapeDtypeStruct((1, 1), jnp.float32),
            jax.ShapeDtypeStruct((_E, _O, _D), jnp.bfloat16),
            jax.ShapeDtypeStruct((_E, _O), jnp.float32),
        ],
        scratch_shapes=[pltpu.VMEM((_KP, _SB, _E), jnp.float32)],
        interpret=interpret,
    )
    moe = pl.pallas_call(
        _moe_body,
        grid=(_O // _OH, _B),
        in_specs=[
            pl.BlockSpec(memory_space=pltpu.SMEM),
            pl.BlockSpec((_N, _T, _D), lambda oh, b: (b, 0, 0)),
            pl.BlockSpec((_E, _OH, _D), lambda oh, b: (0, oh, 0)),
            pl.BlockSpec((_E, _OH), lambda oh, b: (0, oh)),
            pl.BlockSpec((1, _OH), lambda oh, b: (0, oh)),
        ],
        out_specs=pl.BlockSpec((_N, _T, _OH), lambda oh, b: (b, 0, oh)),
        out_shape=jax.ShapeDtypeStruct((_S, _T, _O), jnp.float32),
        interpret=interpret,
    )
    return prep, moe


_PREP, _MOE = _build_calls()


def kernel(x, w_gate, weight, bias, res_weight, res_bias):
    xc = x.reshape(_S, _T, _D)
    gates, loss, wt, dbm = _PREP(xc, w_gate, weight, res_weight,
                                 bias, res_bias)
    out = _MOE(gates, xc, wt, dbm, res_bias)
    return out.reshape(_B, _L, _O), loss[0, 0]
